# baseline (speedup 1.0000x reference)
import functools

import jax
import jax.numpy as jnp
from jax import lax
from jax.experimental import pallas as pl
from jax.experimental.pallas import tpu as pltpu
from jax.experimental.pallas import tpu_sc as plsc

HIST_LEN = 10
ALPHA = 1.0
EPSILON = 0.1
GAMMA = 1.0

ROWS, COLS = 1024, 4096
N_ELEMS = ROWS * COLS

NC, NS, L = 2, 16, 16
NW = NC * NS
PER_W = N_ELEMS // NW
CHUNK = 16384
NCHUNK = PER_W // CHUNK
GROUPS = CHUNK // L


def _exact_thresholds(ranges):
    rk = ranges[:9]
    rk1 = ranges[1:]
    lo = lax.bitcast_convert_type(rk, jnp.int32)
    hi = lax.bitcast_convert_type(rk1, jnp.int32)

    def body(_, lohi):
        lo, hi = lohi
        mid = (lo + hi) // 2
        g = lax.bitcast_convert_type(mid, jnp.float32)
        q = jnp.abs(g - rk1) < jnp.abs(g - rk)
        return jnp.where(q, lo, mid), jnp.where(q, mid, hi)

    lo, hi = lax.fori_loop(0, 32, body, (lo, hi))
    return lax.bitcast_convert_type(hi, jnp.float32)


def _sc_body(pred_hbm, gt_hbm, tab_hbm, out_hbm,
             tlo_v, thi_v, w_v, out_v,
             p0, p1, g0, g1,
             sem_tab, sem_p0, sem_p1, sem_g0, sem_g1, sem_out):
    wid = lax.axis_index("s") * NC + lax.axis_index("c")
    base = wid * PER_W

    pltpu.async_copy(tab_hbm.at[pl.ds(0, L)], tlo_v, sem_tab).wait()
    pltpu.async_copy(tab_hbm.at[pl.ds(L, L)], thi_v, sem_tab).wait()
    pltpu.async_copy(tab_hbm.at[pl.ds(2 * L, L)], w_v, sem_tab).wait()

    tlo_vec = tlo_v[...]
    thi_vec = thi_v[...]
    w_vec = w_v[...]

    pbufs, gbufs = (p0, p1), (g0, g1)
    psems, gsems = (sem_p0, sem_p1), (sem_g0, sem_g1)

    def start(c):
        off = base + c * CHUNK
        hp = pltpu.async_copy(pred_hbm.at[pl.ds(off, CHUNK)], pbufs[c % 2], psems[c % 2])
        hg = pltpu.async_copy(gt_hbm.at[pl.ds(off, CHUNK)], gbufs[c % 2], gsems[c % 2])
        return hp, hg

    handles = {0: start(0)}

    acc_w = jnp.zeros((L,), jnp.float32)
    acc_wd2 = jnp.zeros((L,), jnp.float32)

    for c in range(NCHUNK):
        if c + 1 < NCHUNK:
            handles[c + 1] = start(c + 1)
        hp, hg = handles.pop(c)
        hp.wait()
        hg.wait()
        pbuf, gbuf = pbufs[c % 2], gbufs[c % 2]

        def body(j, carry):
            aw, awd2 = carry
            g = gbuf[pl.ds(j * L, L)]
            p = pbuf[pl.ds(j * L, L)]
            a = g * 9.0 + 0.5
            i0 = a.astype(jnp.int32)
            tlo = tlo_vec.at[i0].get(mode="promise_in_bounds")
            thi = thi_vec.at[i0].get(mode="promise_in_bounds")
            one = jnp.ones((L,), jnp.int32)
            zero = jnp.zeros((L,), jnp.int32)
            i = i0 - jnp.where(g < tlo, one, zero) + jnp.where(g >= thi, one, zero)
            w = w_vec.at[i].get(mode="promise_in_bounds")
            d = g - p
            return aw + w, awd2 + w * (d * d)

        cw, cwd2 = lax.fori_loop(0, GROUPS, body, (jnp.zeros((L,), jnp.float32),
                                                   jnp.zeros((L,), jnp.float32)))
        acc_w = acc_w + cw
        acc_wd2 = acc_wd2 + cwd2

    out_v[0, :] = acc_w
    out_v[1, :] = acc_wd2
    pltpu.async_copy(out_v, out_hbm.at[wid], sem_out).wait()


def kernel(y_pred, y_gt, freqs, ranges):
    ranges = ranges.astype(jnp.float32)
    t = _exact_thresholds(ranges)
    fsum = jnp.sum(freqs).astype(jnp.float32)
    dens = freqs.astype(jnp.float32) / fsum
    wtab = jnp.maximum(1.0 - ALPHA * dens, EPSILON)

    tab = jnp.zeros((3, 16), jnp.float32)
    tab = tab.at[0, 0].set(-1.0).at[0, 1:10].set(t)
    tab = tab.at[1, :9].set(t).at[1, 9:].set(2.0)
    tab = tab.at[2, :10].set(wtab)
    tab = tab.reshape(-1)

    pred_flat = y_pred.reshape(-1)
    gt_flat = y_gt.reshape(-1)

    mesh = plsc.VectorSubcoreMesh(core_axis_name="c", subcore_axis_name="s")
    run = functools.partial(
        pl.kernel,
        out_type=jax.ShapeDtypeStruct((NW, 2, L), jnp.float32),
        mesh=mesh,
        scratch_types=[
            pltpu.VMEM((L,), jnp.float32),
            pltpu.VMEM((L,), jnp.float32),
            pltpu.VMEM((L,), jnp.float32),
            pltpu.VMEM((2, L), jnp.float32),
            pltpu.VMEM((CHUNK,), jnp.float32),
            pltpu.VMEM((CHUNK,), jnp.float32),
            pltpu.VMEM((CHUNK,), jnp.float32),
            pltpu.VMEM((CHUNK,), jnp.float32),
            pltpu.SemaphoreType.DMA,
            pltpu.SemaphoreType.DMA,
            pltpu.SemaphoreType.DMA,
            pltpu.SemaphoreType.DMA,
            pltpu.SemaphoreType.DMA,
            pltpu.SemaphoreType.DMA,
        ],
    )(_sc_body)

    parts = run(pred_flat, gt_flat, tab)
    a = jnp.sum(parts[:, 0, :])
    b = jnp.sum(parts[:, 1, :])
    return GAMMA * jnp.float32(N_ELEMS) * b / a


# speedup vs baseline: 1.0019x; 1.0019x over previous
import functools

import jax
import jax.numpy as jnp
from jax import lax
from jax.experimental import pallas as pl
from jax.experimental.pallas import tpu as pltpu
from jax.experimental.pallas import tpu_sc as plsc

HIST_LEN = 10
ALPHA = 1.0
EPSILON = 0.1
GAMMA = 1.0

ROWS, COLS = 1024, 4096
N_ELEMS = ROWS * COLS

NC, NS, L = 2, 16, 16
NW = NC * NS
PER_W = N_ELEMS // NW
CHUNK = 16384
NCHUNK = PER_W // CHUNK
GROUPS = CHUNK // L


def _exact_thresholds(ranges):
    rk = ranges[:9]
    rk1 = ranges[1:]
    lo = lax.bitcast_convert_type(rk, jnp.int32)
    hi = lax.bitcast_convert_type(rk1, jnp.int32)

    def body(_, lohi):
        lo, hi = lohi
        mid = (lo + hi) // 2
        g = lax.bitcast_convert_type(mid, jnp.float32)
        q = jnp.abs(g - rk1) < jnp.abs(g - rk)
        return jnp.where(q, lo, mid), jnp.where(q, mid, hi)

    lo, hi = lax.fori_loop(0, 32, body, (lo, hi))
    return lax.bitcast_convert_type(hi, jnp.float32)


def _sc_body(pred_hbm, gt_hbm, tab_hbm, out_hbm,
             tlo_v, thi_v, w_v, out_v,
             p0, p1, g0, g1,
             sem_tab, sem_p0, sem_p1, sem_g0, sem_g1, sem_out):
    wid = lax.axis_index("s") * NC + lax.axis_index("c")
    base = wid * PER_W

    pltpu.async_copy(tab_hbm.at[pl.ds(0, L)], tlo_v, sem_tab).wait()
    pltpu.async_copy(tab_hbm.at[pl.ds(L, L)], thi_v, sem_tab).wait()
    pltpu.async_copy(tab_hbm.at[pl.ds(2 * L, L)], w_v, sem_tab).wait()

    tlo_vec = tlo_v[...]
    thi_vec = thi_v[...]
    w_vec = w_v[...]

    pbufs, gbufs = (p0, p1), (g0, g1)
    psems, gsems = (sem_p0, sem_p1), (sem_g0, sem_g1)

    def start(c):
        off = base + c * CHUNK
        hp = pltpu.async_copy(pred_hbm.at[pl.ds(off, CHUNK)], pbufs[c % 2], psems[c % 2])
        hg = pltpu.async_copy(gt_hbm.at[pl.ds(off, CHUNK)], gbufs[c % 2], gsems[c % 2])
        return hp, hg

    handles = {0: start(0)}

    acc_w = jnp.zeros((L,), jnp.float32)
    acc_wd2 = jnp.zeros((L,), jnp.float32)

    for c in range(NCHUNK):
        if c + 1 < NCHUNK:
            handles[c + 1] = start(c + 1)
        hp, hg = handles.pop(c)
        hp.wait()
        hg.wait()
        pbuf, gbuf = pbufs[c % 2], gbufs[c % 2]

        def body(j, carry):
            aw, awd2 = carry
            g = gbuf[pl.ds(j, L)]
            p = pbuf[pl.ds(j, L)]
            a = g * 9.0 + 0.5
            i0 = a.astype(jnp.int32)
            tlo = tlo_vec.at[i0].get(mode="promise_in_bounds")
            thi = thi_vec.at[i0].get(mode="promise_in_bounds")
            one = jnp.ones((L,), jnp.int32)
            zero = jnp.zeros((L,), jnp.int32)
            i = i0 - jnp.where(g < tlo, one, zero) + jnp.where(g >= thi, one, zero)
            w = w_vec.at[i].get(mode="promise_in_bounds")
            d = g - p
            return aw + w, awd2 + w * (d * d)

        cw, cwd2 = plsc.parallel_loop(
            0, CHUNK, step=L, unroll=8,
            carry=(jnp.zeros((L,), jnp.float32), jnp.zeros((L,), jnp.float32)),
        )(body)
        acc_w = acc_w + cw
        acc_wd2 = acc_wd2 + cwd2

    out_v[0, :] = acc_w
    out_v[1, :] = acc_wd2
    pltpu.async_copy(out_v, out_hbm.at[wid], sem_out).wait()


def kernel(y_pred, y_gt, freqs, ranges):
    ranges = ranges.astype(jnp.float32)
    t = _exact_thresholds(ranges)
    fsum = jnp.sum(freqs).astype(jnp.float32)
    dens = freqs.astype(jnp.float32) / fsum
    wtab = jnp.maximum(1.0 - ALPHA * dens, EPSILON)

    tab = jnp.zeros((3, 16), jnp.float32)
    tab = tab.at[0, 0].set(-1.0).at[0, 1:10].set(t)
    tab = tab.at[1, :9].set(t).at[1, 9:].set(2.0)
    tab = tab.at[2, :10].set(wtab)
    tab = tab.reshape(-1)

    pred_flat = y_pred.reshape(-1)
    gt_flat = y_gt.reshape(-1)

    mesh = plsc.VectorSubcoreMesh(core_axis_name="c", subcore_axis_name="s")
    run = functools.partial(
        pl.kernel,
        out_type=jax.ShapeDtypeStruct((NW, 2, L), jnp.float32),
        mesh=mesh,
        scratch_types=[
            pltpu.VMEM((L,), jnp.float32),
            pltpu.VMEM((L,), jnp.float32),
            pltpu.VMEM((L,), jnp.float32),
            pltpu.VMEM((2, L), jnp.float32),
            pltpu.VMEM((CHUNK,), jnp.float32),
            pltpu.VMEM((CHUNK,), jnp.float32),
            pltpu.VMEM((CHUNK,), jnp.float32),
            pltpu.VMEM((CHUNK,), jnp.float32),
            pltpu.SemaphoreType.DMA,
            pltpu.SemaphoreType.DMA,
            pltpu.SemaphoreType.DMA,
            pltpu.SemaphoreType.DMA,
            pltpu.SemaphoreType.DMA,
            pltpu.SemaphoreType.DMA,
        ],
    )(_sc_body)

    parts = run(pred_flat, gt_flat, tab)
    a = jnp.sum(parts[:, 0, :])
    b = jnp.sum(parts[:, 1, :])
    return GAMMA * jnp.float32(N_ELEMS) * b / a


# speedup vs baseline: 1.0534x; 1.0514x over previous
import functools

import jax
import jax.numpy as jnp
from jax import lax
from jax.experimental import pallas as pl
from jax.experimental.pallas import tpu as pltpu
from jax.experimental.pallas import tpu_sc as plsc

HIST_LEN = 10
ALPHA = 1.0
EPSILON = 0.1
GAMMA = 1.0

ROWS, COLS = 1024, 4096
N_ELEMS = ROWS * COLS

NC, NS, L = 2, 16, 16
NW = NC * NS
SC_ROWS = 320
SC_N = SC_ROWS * COLS
PER_W = SC_N // NW
NCHUNK = 5
CHUNK = PER_W // NCHUNK

TC_ROWS = ROWS - SC_ROWS
TC_GRID = 4
TC_BLOCK_ROWS = TC_ROWS // TC_GRID


def _exact_thresholds(ranges):
    rk = ranges[:9]
    rk1 = ranges[1:]
    lo = lax.bitcast_convert_type(rk, jnp.int32)
    hi = lax.bitcast_convert_type(rk1, jnp.int32)

    def body(_, lohi):
        lo, hi = lohi
        mid = (lo + hi) // 2
        g = lax.bitcast_convert_type(mid, jnp.float32)
        q = jnp.abs(g - rk1) < jnp.abs(g - rk)
        return jnp.where(q, lo, mid), jnp.where(q, mid, hi)

    lo, hi = lax.fori_loop(0, 32, body, (lo, hi))
    return lax.bitcast_convert_type(hi, jnp.float32)


def _sc_body(pred_hbm, gt_hbm, tab_hbm, out_hbm,
             tlo_v, thi_v, w_v, out_v,
             p0, p1, g0, g1,
             sem_tab, sem_p0, sem_p1, sem_g0, sem_g1, sem_out):
    wid = lax.axis_index("s") * NC + lax.axis_index("c")
    base = wid * PER_W

    pltpu.async_copy(tab_hbm.at[pl.ds(0, L)], tlo_v, sem_tab).wait()
    pltpu.async_copy(tab_hbm.at[pl.ds(L, L)], thi_v, sem_tab).wait()
    pltpu.async_copy(tab_hbm.at[pl.ds(2 * L, L)], w_v, sem_tab).wait()

    tlo_vec = tlo_v[...]
    thi_vec = thi_v[...]
    w_vec = w_v[...]

    pbufs, gbufs = (p0, p1), (g0, g1)
    psems, gsems = (sem_p0, sem_p1), (sem_g0, sem_g1)

    def start(c):
        off = base + c * CHUNK
        hp = pltpu.async_copy(pred_hbm.at[pl.ds(off, CHUNK)], pbufs[c % 2], psems[c % 2])
        hg = pltpu.async_copy(gt_hbm.at[pl.ds(off, CHUNK)], gbufs[c % 2], gsems[c % 2])
        return hp, hg

    handles = {0: start(0)}

    acc_w = jnp.zeros((L,), jnp.float32)
    acc_wd2 = jnp.zeros((L,), jnp.float32)

    for c in range(NCHUNK):
        if c + 1 < NCHUNK:
            handles[c + 1] = start(c + 1)
        hp, hg = handles.pop(c)
        hp.wait()
        hg.wait()
        pbuf, gbuf = pbufs[c % 2], gbufs[c % 2]

        def body(j, carry):
            aw, awd2 = carry
            g = gbuf[pl.ds(j, L)]
            p = pbuf[pl.ds(j, L)]
            a = g * 9.0 + 0.5
            i0 = a.astype(jnp.int32)
            tlo = tlo_vec.at[i0].get(mode="promise_in_bounds")
            thi = thi_vec.at[i0].get(mode="promise_in_bounds")
            one = jnp.ones((L,), jnp.int32)
            zero = jnp.zeros((L,), jnp.int32)
            i = i0 - jnp.where(g < tlo, one, zero) + jnp.where(g >= thi, one, zero)
            w = w_vec.at[i].get(mode="promise_in_bounds")
            d = g - p
            return aw + w, awd2 + w * (d * d)

        cw, cwd2 = plsc.parallel_loop(
            0, CHUNK, step=L, unroll=8,
            carry=(jnp.zeros((L,), jnp.float32), jnp.zeros((L,), jnp.float32)),
        )(body)
        acc_w = acc_w + cw
        acc_wd2 = acc_wd2 + cwd2

    out_v[0, :] = acc_w
    out_v[1, :] = acc_wd2
    pltpu.async_copy(out_v, out_hbm.at[wid], sem_out).wait()


def _tc_body(scal_ref, pred_ref, gt_ref, ow_ref, owd2_ref):
    @pl.when(pl.program_id(0) == 0)
    def _():
        ow_ref[0, 0] = 0.0
        owd2_ref[0, 0] = 0.0

    g = gt_ref[...]
    p = pred_ref[...]
    d = g - p
    d2 = d * d
    w = jnp.full_like(g, scal_ref[1, 15])
    for k in range(9):
        w = w + jnp.where(g >= scal_ref[0, k], scal_ref[1, k], 0.0)
    ow_ref[0, 0] += jnp.sum(w)
    owd2_ref[0, 0] += jnp.sum(w * d2)


def kernel(y_pred, y_gt, freqs, ranges):
    ranges = ranges.astype(jnp.float32)
    t = _exact_thresholds(ranges)
    fsum = jnp.sum(freqs).astype(jnp.float32)
    dens = freqs.astype(jnp.float32) / fsum
    wtab = jnp.maximum(1.0 - ALPHA * dens, EPSILON)
    deltas = wtab[1:] - wtab[:9]

    tab = jnp.zeros((3, 16), jnp.float32)
    tab = tab.at[0, 0].set(-1.0).at[0, 1:10].set(t)
    tab = tab.at[1, :9].set(t).at[1, 9:].set(2.0)
    tab = tab.at[2, :10].set(wtab)
    tab = tab.reshape(-1)

    scal = jnp.zeros((2, 16), jnp.float32)
    scal = scal.at[0, :9].set(t).at[0, 9:].set(9e9)
    scal = scal.at[1, :9].set(deltas).at[1, 15].set(wtab[0])

    pred_flat = y_pred.reshape(-1)[:SC_N]
    gt_flat = y_gt.reshape(-1)[:SC_N]

    mesh = plsc.VectorSubcoreMesh(core_axis_name="c", subcore_axis_name="s")
    sc_run = functools.partial(
        pl.kernel,
        out_type=jax.ShapeDtypeStruct((NW, 2, L), jnp.float32),
        mesh=mesh,
        scratch_types=[
            pltpu.VMEM((L,), jnp.float32),
            pltpu.VMEM((L,), jnp.float32),
            pltpu.VMEM((L,), jnp.float32),
            pltpu.VMEM((2, L), jnp.float32),
            pltpu.VMEM((CHUNK,), jnp.float32),
            pltpu.VMEM((CHUNK,), jnp.float32),
            pltpu.VMEM((CHUNK,), jnp.float32),
            pltpu.VMEM((CHUNK,), jnp.float32),
            pltpu.SemaphoreType.DMA,
            pltpu.SemaphoreType.DMA,
            pltpu.SemaphoreType.DMA,
            pltpu.SemaphoreType.DMA,
            pltpu.SemaphoreType.DMA,
            pltpu.SemaphoreType.DMA,
        ],
    )(_sc_body)

    sc_parts = sc_run(pred_flat, gt_flat, tab)

    tc_ow, tc_owd2 = pl.pallas_call(
        _tc_body,
        grid=(TC_GRID,),
        in_specs=[
            pl.BlockSpec(memory_space=pltpu.SMEM),
            pl.BlockSpec((TC_BLOCK_ROWS, COLS), lambda i: (i, 0)),
            pl.BlockSpec((TC_BLOCK_ROWS, COLS), lambda i: (i, 0)),
        ],
        out_specs=[
            pl.BlockSpec(memory_space=pltpu.SMEM),
            pl.BlockSpec(memory_space=pltpu.SMEM),
        ],
        out_shape=[
            jax.ShapeDtypeStruct((1, 1), jnp.float32),
            jax.ShapeDtypeStruct((1, 1), jnp.float32),
        ],
    )(scal, y_pred[SC_ROWS:], y_gt[SC_ROWS:])

    a = jnp.sum(sc_parts[:, 0, :]) + tc_ow[0, 0]
    b = jnp.sum(sc_parts[:, 1, :]) + tc_owd2[0, 0]
    return GAMMA * jnp.float32(N_ELEMS) * b / a
